# baseline (speedup 1.0000x reference)
import functools

import jax
import jax.numpy as jnp
from jax import lax
from jax.experimental import pallas as pl
from jax.experimental.pallas import tpu as pltpu
from jax.experimental.pallas import tpu_sc as plsc

N = 10000
E = 320000
DIN = 128
DH = 128
DOUT = 64

NC = 2
NS = 16
NW = NC * NS
CHUNK = 128

NACC = 10240
ROWS_PER_TILE = NACC // NS
ZR = 128

EF = E + N
CPT = 82
EPT = CPT * CHUNK
EP = EPT * NW
NCHUNK = EP // CHUNK

ROW_BLOCK = 1024
GRID = (N + ROW_BLOCK - 1) // ROW_BLOCK

_mesh = plsc.VectorSubcoreMesh(core_axis_name="c", subcore_axis_name="s")
_sc_params = pltpu.CompilerParams(use_tc_tiling_on_sc=False)


def _sc_degree(dst_chunks):
    @functools.partial(
        pl.kernel,
        out_type=jax.ShapeDtypeStruct((NC, NACC), jnp.float32),
        mesh=_mesh,
        compiler_params=_sc_params,
        scratch_types=[
            pltpu.VMEM_SHARED((NACC,), jnp.float32),
            pltpu.VMEM((CPT, CHUNK), jnp.int32),
            pltpu.VMEM((CHUNK,), jnp.float32),
            pltpu.VMEM((ROWS_PER_TILE,), jnp.float32),
        ],
    )
    def deg_kernel(dst_hbm, out_hbm, acc_sh, idx_v, ones_v, zeros_v):
        c = lax.axis_index("c")
        s = lax.axis_index("s")
        wid = c * NS + s

        @pl.loop(0, ROWS_PER_TILE, step=16)
        def _(i):
            zeros_v[pl.ds(i, 16)] = jnp.zeros((16,), jnp.float32)

        @pl.loop(0, CHUNK, step=16)
        def _(i):
            ones_v[pl.ds(i, 16)] = jnp.ones((16,), jnp.float32)

        row0 = s * ROWS_PER_TILE
        pltpu.sync_copy(zeros_v, acc_sh.at[pl.ds(row0, ROWS_PER_TILE)])
        pltpu.sync_copy(dst_hbm.at[pl.ds(wid * CPT, CPT)], idx_v)
        plsc.subcore_barrier()

        @pl.loop(0, CPT)
        def _(ch):
            pltpu.sync_copy(ones_v, acc_sh.at[idx_v.at[ch]], add=True)

        plsc.subcore_barrier()
        pltpu.sync_copy(acc_sh.at[pl.ds(row0, ROWS_PER_TILE)],
                        out_hbm.at[c, pl.ds(row0, ROWS_PER_TILE)])

    return deg_kernel(dst_chunks)


def _sc_aggregate(src_chunks, dst_chunks, y, d):
    @functools.partial(
        pl.kernel,
        out_type=jax.ShapeDtypeStruct((NC, NACC, d), jnp.float32),
        mesh=_mesh,
        compiler_params=_sc_params,
        scratch_types=[
            pltpu.VMEM_SHARED((NACC, d), jnp.float32),
            pltpu.VMEM((CPT, CHUNK), jnp.int32),
            pltpu.VMEM((CHUNK,), jnp.int32),
            pltpu.VMEM((CHUNK,), jnp.int32),
            pltpu.VMEM((CHUNK, d), jnp.float32),
            pltpu.VMEM((CHUNK, d), jnp.float32),
            pltpu.SemaphoreType.DMA,
            pltpu.SemaphoreType.DMA,
            pltpu.SemaphoreType.DMA,
            pltpu.SemaphoreType.DMA,
        ],
    )
    def agg_kernel(src_hbm, dst_hbm, y_hbm, out_hbm,
                   acc_sh, src_v, dst0, dst1, rows0, rows1,
                   sg0, sg1, sd0, sd1):
        c = lax.axis_index("c")
        s = lax.axis_index("s")
        wid = c * NS + s

        @pl.loop(0, CHUNK)
        def _(r):
            for j in range(d // 16):
                rows0[r, pl.ds(j * 16, 16)] = jnp.zeros((16,), jnp.float32)

        row0 = s * ROWS_PER_TILE

        @pl.loop(0, ROWS_PER_TILE, step=CHUNK)
        def _(r):
            pltpu.sync_copy(rows0, acc_sh.at[pl.ds(row0 + r, CHUNK)])

        crow0 = wid * CPT
        pltpu.sync_copy(src_hbm.at[pl.ds(crow0, CPT)], src_v)
        plsc.subcore_barrier()

        pltpu.async_copy(y_hbm.at[src_v.at[0]], rows0, sg0)
        pltpu.async_copy(dst_hbm.at[crow0], dst0, sd0)

        @pl.loop(0, CPT, step=2)
        def _(ch):
            crow = crow0 + ch
            pltpu.async_copy(y_hbm.at[src_v.at[ch + 1]], rows1, sg1)
            pltpu.async_copy(dst_hbm.at[crow + 1], dst1, sd1)
            pltpu.make_async_copy(y_hbm.at[src_v.at[ch]], rows0, sg0).wait()
            pltpu.make_async_copy(dst_hbm.at[crow], dst0, sd0).wait()
            pltpu.sync_copy(rows0, acc_sh.at[dst0], add=True)

            @pl.when(ch + 2 < CPT)
            def _():
                pltpu.async_copy(y_hbm.at[src_v.at[ch + 2]], rows0, sg0)
                pltpu.async_copy(dst_hbm.at[crow + 2], dst0, sd0)

            pltpu.make_async_copy(y_hbm.at[src_v.at[ch + 1]], rows1,
                                  sg1).wait()
            pltpu.make_async_copy(dst_hbm.at[crow + 1], dst1, sd1).wait()
            pltpu.sync_copy(rows1, acc_sh.at[dst1], add=True)

        plsc.subcore_barrier()
        pltpu.sync_copy(acc_sh.at[pl.ds(row0, ROWS_PER_TILE)],
                        out_hbm.at[c, pl.ds(row0, ROWS_PER_TILE)])

    return agg_kernel(src_chunks, dst_chunks, y)


def _tc_matmul(x, w):
    dout = w.shape[1]

    def body(x_ref, w_ref, o_ref):
        o_ref[...] = jnp.dot(x_ref[...], w_ref[...],
                             preferred_element_type=jnp.float32)

    return pl.pallas_call(
        body,
        grid=(GRID,),
        in_specs=[
            pl.BlockSpec((ROW_BLOCK, x.shape[1]), lambda i: (i, 0)),
            pl.BlockSpec((w.shape[0], dout), lambda i: (0, 0)),
        ],
        out_specs=pl.BlockSpec((ROW_BLOCK, dout), lambda i: (i, 0)),
        out_shape=jax.ShapeDtypeStruct((N, dout), jnp.float32),
    )(x, w)


def _tc_scale(xw, deg_a, deg_b):
    def body(xw_ref, da_ref, db_ref, y_ref, dinv_ref):
        dinv = lax.rsqrt(jnp.maximum(da_ref[...] + db_ref[...], 1e-12))
        dinv_ref[...] = dinv
        y_ref[...] = xw_ref[...] * dinv

    return pl.pallas_call(
        body,
        grid=(GRID,),
        in_specs=[
            pl.BlockSpec((ROW_BLOCK, DH), lambda i: (i, 0)),
            pl.BlockSpec((ROW_BLOCK, 1), lambda i: (i, 0)),
            pl.BlockSpec((ROW_BLOCK, 1), lambda i: (i, 0)),
        ],
        out_specs=[
            pl.BlockSpec((ROW_BLOCK, DH), lambda i: (i, 0)),
            pl.BlockSpec((ROW_BLOCK, 1), lambda i: (i, 0)),
        ],
        out_shape=[
            jax.ShapeDtypeStruct((N, DH), jnp.float32),
            jax.ShapeDtypeStruct((N, 1), jnp.float32),
        ],
    )(xw, deg_a, deg_b)


def _tc_layer2_in(s_a, s_b, dinv, b1, w2):
    def body(sa_ref, sb_ref, dinv_ref, b1_ref, w2_ref, y2_ref):
        dinv = dinv_ref[...]
        h = jnp.maximum(dinv * (sa_ref[...] + sb_ref[...]) + b1_ref[...], 0.0)
        y2_ref[...] = dinv * jnp.dot(h, w2_ref[...],
                                     preferred_element_type=jnp.float32)

    return pl.pallas_call(
        body,
        grid=(GRID,),
        in_specs=[
            pl.BlockSpec((ROW_BLOCK, DH), lambda i: (i, 0)),
            pl.BlockSpec((ROW_BLOCK, DH), lambda i: (i, 0)),
            pl.BlockSpec((ROW_BLOCK, 1), lambda i: (i, 0)),
            pl.BlockSpec((1, DH), lambda i: (0, 0)),
            pl.BlockSpec((DH, DOUT), lambda i: (0, 0)),
        ],
        out_specs=pl.BlockSpec((ROW_BLOCK, DOUT), lambda i: (i, 0)),
        out_shape=jax.ShapeDtypeStruct((N, DOUT), jnp.float32),
    )(s_a, s_b, dinv, b1, w2)


def _tc_final(s_a, s_b, dinv, b2):
    def body(sa_ref, sb_ref, dinv_ref, b2_ref, o_ref):
        o_ref[...] = (dinv_ref[...] * (sa_ref[...] + sb_ref[...])
                      + b2_ref[...])

    return pl.pallas_call(
        body,
        grid=(GRID,),
        in_specs=[
            pl.BlockSpec((ROW_BLOCK, DOUT), lambda i: (i, 0)),
            pl.BlockSpec((ROW_BLOCK, DOUT), lambda i: (i, 0)),
            pl.BlockSpec((ROW_BLOCK, 1), lambda i: (i, 0)),
            pl.BlockSpec((1, DOUT), lambda i: (0, 0)),
        ],
        out_specs=pl.BlockSpec((ROW_BLOCK, DOUT), lambda i: (i, 0)),
        out_shape=jax.ShapeDtypeStruct((N, DOUT), jnp.float32),
    )(s_a, s_b, dinv, b2)


def kernel(x, edge_index, W1, b1, W2, b2):
    src = edge_index[0]
    dst = edge_index[1]
    loop = jnp.arange(N, dtype=jnp.int32)
    pad = EP - EF
    src_full = jnp.concatenate([src, loop, jnp.zeros((pad,), jnp.int32)])
    dst_full = jnp.concatenate([dst, loop, jnp.full((pad,), N, jnp.int32)])
    src_chunks = src_full.reshape(NCHUNK, CHUNK)
    dst_chunks = dst_full.reshape(NCHUNK, CHUNK)

    deg_p = _sc_degree(dst_chunks)
    deg_a = deg_p[0].reshape(NACC, 1)
    deg_b = deg_p[1].reshape(NACC, 1)

    xw1 = _tc_matmul(x, W1)
    y1, dinv = _tc_scale(xw1, deg_a, deg_b)

    s1 = _sc_aggregate(src_chunks, dst_chunks, y1, DH)
    y2 = _tc_layer2_in(s1[0], s1[1], dinv,
                       b1.reshape(1, DH), W2)

    s2 = _sc_aggregate(src_chunks, dst_chunks, y2, DOUT)
    return _tc_final(s2[0], s2[1], dinv, b2.reshape(1, DOUT))


# speedup vs baseline: 1.5061x; 1.5061x over previous
import functools

import jax
import jax.numpy as jnp
from jax import lax
from jax.experimental import pallas as pl
from jax.experimental.pallas import tpu as pltpu
from jax.experimental.pallas import tpu_sc as plsc

N = 10000
E = 320000
DIN = 128
DH = 128
DOUT = 64

NC = 2
NS = 16
NW = NC * NS
CHUNK = 128

NACC = 10240
ROWS_PER_TILE = NACC // NS
ZR = 128

EF = E + N
CPT = 82
EPT = CPT * CHUNK
EP = EPT * NW
NCHUNK = EP // CHUNK
CPTA = NCHUNK // NS

ROW_BLOCK = 1024
GRID = (N + ROW_BLOCK - 1) // ROW_BLOCK

_mesh = plsc.VectorSubcoreMesh(core_axis_name="c", subcore_axis_name="s")
_sc_params = pltpu.CompilerParams(use_tc_tiling_on_sc=False)


def _sc_degree(dst_chunks):
    @functools.partial(
        pl.kernel,
        out_type=jax.ShapeDtypeStruct((NC, NACC), jnp.float32),
        mesh=_mesh,
        compiler_params=_sc_params,
        scratch_types=[
            pltpu.VMEM_SHARED((NACC,), jnp.float32),
            pltpu.VMEM((CPT, CHUNK), jnp.int32),
            pltpu.VMEM((CHUNK,), jnp.float32),
            pltpu.VMEM((ROWS_PER_TILE,), jnp.float32),
        ],
    )
    def deg_kernel(dst_hbm, out_hbm, acc_sh, idx_v, ones_v, zeros_v):
        c = lax.axis_index("c")
        s = lax.axis_index("s")
        wid = c * NS + s

        @pl.loop(0, ROWS_PER_TILE, step=16)
        def _(i):
            zeros_v[pl.ds(i, 16)] = jnp.zeros((16,), jnp.float32)

        @pl.loop(0, CHUNK, step=16)
        def _(i):
            ones_v[pl.ds(i, 16)] = jnp.ones((16,), jnp.float32)

        row0 = s * ROWS_PER_TILE
        pltpu.sync_copy(zeros_v, acc_sh.at[pl.ds(row0, ROWS_PER_TILE)])
        pltpu.sync_copy(dst_hbm.at[pl.ds(wid * CPT, CPT)], idx_v)
        plsc.subcore_barrier()

        @pl.loop(0, CPT)
        def _(ch):
            pltpu.sync_copy(ones_v, acc_sh.at[idx_v.at[ch]], add=True)

        plsc.subcore_barrier()
        pltpu.sync_copy(acc_sh.at[pl.ds(row0, ROWS_PER_TILE)],
                        out_hbm.at[c, pl.ds(row0, ROWS_PER_TILE)])

    return deg_kernel(dst_chunks)


def _sc_aggregate(src_chunks, dst_chunks, y_pair, dh):
    @functools.partial(
        pl.kernel,
        out_type=jax.ShapeDtypeStruct((NC, NACC, dh), jnp.float32),
        mesh=_mesh,
        compiler_params=_sc_params,
        scratch_types=[
            pltpu.VMEM_SHARED((NACC, dh), jnp.float32),
            pltpu.VMEM((CPTA, CHUNK), jnp.int32),
            pltpu.VMEM((CHUNK,), jnp.int32),
            pltpu.VMEM((CHUNK,), jnp.int32),
            pltpu.VMEM((CHUNK, dh), jnp.float32),
            pltpu.VMEM((CHUNK, dh), jnp.float32),
            pltpu.SemaphoreType.DMA,
            pltpu.SemaphoreType.DMA,
            pltpu.SemaphoreType.DMA,
            pltpu.SemaphoreType.DMA,
        ],
    )
    def agg_kernel(src_hbm, dst_hbm, y_hbm, out_hbm,
                   acc_sh, src_v, dst0, dst1, rows0, rows1,
                   sg0, sg1, sd0, sd1):
        c = lax.axis_index("c")
        s = lax.axis_index("s")
        y_c = y_hbm.at[c]

        @pl.loop(0, CHUNK)
        def _(r):
            for j in range(dh // 16):
                rows0[r, pl.ds(j * 16, 16)] = jnp.zeros((16,), jnp.float32)

        row0 = s * ROWS_PER_TILE

        @pl.loop(0, ROWS_PER_TILE, step=CHUNK)
        def _(r):
            pltpu.sync_copy(rows0, acc_sh.at[pl.ds(row0 + r, CHUNK)])

        crow0 = s * CPTA
        pltpu.sync_copy(src_hbm.at[pl.ds(crow0, CPTA)], src_v)
        plsc.subcore_barrier()

        pltpu.async_copy(y_c.at[src_v.at[0]], rows0, sg0)
        pltpu.async_copy(dst_hbm.at[crow0], dst0, sd0)

        @pl.loop(0, CPTA, step=2)
        def _(ch):
            crow = crow0 + ch
            pltpu.async_copy(y_c.at[src_v.at[ch + 1]], rows1, sg1)
            pltpu.async_copy(dst_hbm.at[crow + 1], dst1, sd1)
            pltpu.make_async_copy(y_c.at[src_v.at[ch]], rows0, sg0).wait()
            pltpu.make_async_copy(dst_hbm.at[crow], dst0, sd0).wait()
            pltpu.sync_copy(rows0, acc_sh.at[dst0], add=True)

            @pl.when(ch + 2 < CPTA)
            def _():
                pltpu.async_copy(y_c.at[src_v.at[ch + 2]], rows0, sg0)
                pltpu.async_copy(dst_hbm.at[crow + 2], dst0, sd0)

            pltpu.make_async_copy(y_c.at[src_v.at[ch + 1]], rows1,
                                  sg1).wait()
            pltpu.make_async_copy(dst_hbm.at[crow + 1], dst1, sd1).wait()
            pltpu.sync_copy(rows1, acc_sh.at[dst1], add=True)

        plsc.subcore_barrier()
        pltpu.sync_copy(acc_sh.at[pl.ds(row0, ROWS_PER_TILE)],
                        out_hbm.at[c, pl.ds(row0, ROWS_PER_TILE)])

    return agg_kernel(src_chunks, dst_chunks, y_pair)


def _tc_matmul(x, w):
    dout = w.shape[1]

    def body(x_ref, w_ref, o_ref):
        o_ref[...] = jnp.dot(x_ref[...], w_ref[...],
                             preferred_element_type=jnp.float32)

    return pl.pallas_call(
        body,
        grid=(GRID,),
        in_specs=[
            pl.BlockSpec((ROW_BLOCK, x.shape[1]), lambda i: (i, 0)),
            pl.BlockSpec((w.shape[0], dout), lambda i: (0, 0)),
        ],
        out_specs=pl.BlockSpec((ROW_BLOCK, dout), lambda i: (i, 0)),
        out_shape=jax.ShapeDtypeStruct((N, dout), jnp.float32),
    )(x, w)


def _tc_scale(xw, deg_a, deg_b):
    dh = DH // 2

    def body(xw_ref, da_ref, db_ref, y_ref, dinv_ref):
        dinv = lax.rsqrt(jnp.maximum(da_ref[...] + db_ref[...], 1e-12))
        dinv_ref[...] = dinv
        y = xw_ref[...] * dinv
        y_ref[0] = y[:, :dh]
        y_ref[1] = y[:, dh:]

    return pl.pallas_call(
        body,
        grid=(GRID,),
        in_specs=[
            pl.BlockSpec((ROW_BLOCK, DH), lambda i: (i, 0)),
            pl.BlockSpec((ROW_BLOCK, 1), lambda i: (i, 0)),
            pl.BlockSpec((ROW_BLOCK, 1), lambda i: (i, 0)),
        ],
        out_specs=[
            pl.BlockSpec((2, ROW_BLOCK, dh), lambda i: (0, i, 0)),
            pl.BlockSpec((ROW_BLOCK, 1), lambda i: (i, 0)),
        ],
        out_shape=[
            jax.ShapeDtypeStruct((2, N, dh), jnp.float32),
            jax.ShapeDtypeStruct((N, 1), jnp.float32),
        ],
    )(xw, deg_a, deg_b)


def _tc_layer2_in(s1, dinv, b1, w2):
    dh = DH // 2
    do2 = DOUT // 2

    def body(s_ref, dinv_ref, b1_ref, w2_ref, y2_ref):
        dinv = dinv_ref[...]
        b1 = b1_ref[...]
        w2 = w2_ref[...]
        ha = jnp.maximum(dinv * s_ref[0] + b1[:, :dh], 0.0)
        hb = jnp.maximum(dinv * s_ref[1] + b1[:, dh:], 0.0)
        y2 = dinv * (jnp.dot(ha, w2[:dh], preferred_element_type=jnp.float32)
                     + jnp.dot(hb, w2[dh:],
                               preferred_element_type=jnp.float32))
        y2_ref[0] = y2[:, :do2]
        y2_ref[1] = y2[:, do2:]

    return pl.pallas_call(
        body,
        grid=(GRID,),
        in_specs=[
            pl.BlockSpec((2, ROW_BLOCK, dh), lambda i: (0, i, 0)),
            pl.BlockSpec((ROW_BLOCK, 1), lambda i: (i, 0)),
            pl.BlockSpec((1, DH), lambda i: (0, 0)),
            pl.BlockSpec((DH, DOUT), lambda i: (0, 0)),
        ],
        out_specs=pl.BlockSpec((2, ROW_BLOCK, do2), lambda i: (0, i, 0)),
        out_shape=jax.ShapeDtypeStruct((2, N, do2), jnp.float32),
    )(s1, dinv, b1, w2)


def _tc_final(s2, dinv, b2):
    do2 = DOUT // 2

    def body(s_ref, dinv_ref, b2_ref, o_ref):
        dinv = dinv_ref[...]
        b2 = b2_ref[...]
        o_ref[:, :do2] = dinv * s_ref[0] + b2[:, :do2]
        o_ref[:, do2:] = dinv * s_ref[1] + b2[:, do2:]

    return pl.pallas_call(
        body,
        grid=(GRID,),
        in_specs=[
            pl.BlockSpec((2, ROW_BLOCK, do2), lambda i: (0, i, 0)),
            pl.BlockSpec((ROW_BLOCK, 1), lambda i: (i, 0)),
            pl.BlockSpec((1, DOUT), lambda i: (0, 0)),
        ],
        out_specs=pl.BlockSpec((ROW_BLOCK, DOUT), lambda i: (i, 0)),
        out_shape=jax.ShapeDtypeStruct((N, DOUT), jnp.float32),
    )(s2, dinv, b2)


def kernel(x, edge_index, W1, b1, W2, b2):
    src = edge_index[0]
    dst = edge_index[1]
    loop = jnp.arange(N, dtype=jnp.int32)
    pad = EP - EF
    src_full = jnp.concatenate([src, loop, jnp.zeros((pad,), jnp.int32)])
    dst_full = jnp.concatenate([dst, loop, jnp.full((pad,), N, jnp.int32)])
    src_chunks = src_full.reshape(NCHUNK, CHUNK)
    dst_chunks = dst_full.reshape(NCHUNK, CHUNK)

    deg_p = _sc_degree(dst_chunks)
    deg_a = deg_p[0].reshape(NACC, 1)
    deg_b = deg_p[1].reshape(NACC, 1)

    xw1 = _tc_matmul(x, W1)
    y1p, dinv = _tc_scale(xw1, deg_a, deg_b)

    s1 = _sc_aggregate(src_chunks, dst_chunks, y1p, DH // 2)
    y2p = _tc_layer2_in(s1, dinv, b1.reshape(1, DH), W2)

    s2 = _sc_aggregate(src_chunks, dst_chunks, y2p, DOUT // 2)
    return _tc_final(s2, dinv, b2.reshape(1, DOUT))
